# baseline (device time: 28916 ns/iter reference)
import jax
import jax.numpy as jnp
from jax import lax
from jax.experimental import pallas as pl
from jax.experimental.pallas import tpu as pltpu

N_DEV = 4
N_LOCAL_E = 4
N_EXPERTS = 16
CAPACITY = 51
CAP_P = 56
SLOTS = N_LOCAL_E * CAP_P
G_SLOTS = N_DEV * SLOTS
N_TOK = 1024
D_IN = 512
D_OUT = 1024


def kernel(x, router_W, route_idx, expert_W):
    del router_W

    def body(x_ref, r_ref, w_ref, out_ref, comm_ref, y_ref, send_sems, recv_sems):
        my_pos = lax.axis_index("i")
        left = (my_pos - 1) % N_DEV
        right = (my_pos + 1) % N_DEV
        diag = (my_pos + 2) % N_DEV
        peers = [left, right, diag]

        barrier_sem = pltpu.get_barrier_semaphore()
        for nbr in peers:
            pl.semaphore_signal(
                barrier_sem, inc=1,
                device_id=(nbr,), device_id_type=pl.DeviceIdType.MESH,
            )
        pl.semaphore_wait(barrier_sem, len(peers))

        r = r_ref[...]
        e_iota = lax.broadcasted_iota(jnp.int32, (N_TOK, N_EXPERTS), 1)
        onehot = (r == e_iota).astype(jnp.float32)
        row_i = lax.broadcasted_iota(jnp.int32, (N_TOK, N_TOK), 0)
        col_i = lax.broadcasted_iota(jnp.int32, (N_TOK, N_TOK), 1)
        lower_tri = (row_i >= col_i).astype(jnp.float32)
        cum = jnp.dot(lower_tri, onehot, preferred_element_type=jnp.float32)
        rank = jnp.sum(onehot * cum, axis=1, keepdims=True) - 1.0
        keep = rank < CAPACITY
        slot = r.astype(jnp.float32) * CAP_P + rank

        col_me = (
            my_pos * SLOTS
            + lax.broadcasted_iota(jnp.int32, (N_TOK, SLOTS), 1)
        ).astype(jnp.float32)
        s_me = ((slot == col_me) & keep).astype(jnp.float32)
        xg = lax.dot_general(
            s_me, x_ref[...],
            dimension_numbers=(((0,), (0,)), ((), ())),
            preferred_element_type=jnp.float32,
        )
        y = jnp.concatenate(
            [
                jnp.dot(
                    xg[j * CAP_P : (j + 1) * CAP_P, :],
                    w_ref[j],
                    preferred_element_type=jnp.float32,
                )
                for j in range(N_LOCAL_E)
            ],
            axis=0,
        ).astype(jnp.bfloat16)
        y_ref[...] = y
        comm_ref[my_pos] = y

        sends = []
        for t in peers:
            rdma = pltpu.make_async_remote_copy(
                src_ref=y_ref,
                dst_ref=comm_ref.at[my_pos],
                send_sem=send_sems.at[t],
                recv_sem=recv_sems.at[my_pos],
                device_id=(t,), device_id_type=pl.DeviceIdType.MESH,
            )
            rdma.start()
            sends.append(rdma)

        col_all = lax.broadcasted_iota(
            jnp.int32, (N_TOK, G_SLOTS), 1
        ).astype(jnp.float32)
        s_all = ((slot == col_all) & keep).astype(jnp.bfloat16)

        for s in peers:
            pltpu.make_async_remote_copy(
                src_ref=y_ref,
                dst_ref=comm_ref.at[s],
                send_sem=send_sems.at[s],
                recv_sem=recv_sems.at[s],
                device_id=(s,), device_id_type=pl.DeviceIdType.MESH,
            ).wait_recv()

        y_all = comm_ref[...].reshape(G_SLOTS, D_OUT)
        out_ref[...] = jnp.dot(
            s_all, y_all, preferred_element_type=jnp.float32
        )

        for rdma in sends:
            rdma.wait_send()

    return pl.pallas_call(
        body,
        out_shape=jax.ShapeDtypeStruct((N_TOK, D_OUT), jnp.float32),
        in_specs=[
            pl.BlockSpec(memory_space=pltpu.VMEM),
            pl.BlockSpec(memory_space=pltpu.VMEM),
            pl.BlockSpec(memory_space=pltpu.VMEM),
        ],
        out_specs=pl.BlockSpec(memory_space=pltpu.VMEM),
        scratch_shapes=[
            pltpu.VMEM((N_DEV, SLOTS, D_OUT), jnp.bfloat16),
            pltpu.VMEM((SLOTS, D_OUT), jnp.bfloat16),
            pltpu.SemaphoreType.DMA((N_DEV,)),
            pltpu.SemaphoreType.DMA((N_DEV,)),
        ],
        compiler_params=pltpu.CompilerParams(collective_id=0),
    )(x, route_idx, expert_W)
